# Initial kernel scaffold; baseline (speedup 1.0000x reference)
import jax
import jax.numpy as jnp
from jax.experimental import pallas as pl


def kernel(x, edge_index, edge_attr, W_msg1, b_msg1, W_msg2, b_msg2, W_udt1, b_udt1, W_udt2, b_udt2):
    raise NotImplementedError("write your pallas kernel here")


# speedup vs baseline: 2.4348x; 2.4348x over previous
import functools

import jax
import jax.numpy as jnp
from jax import lax
from jax.experimental import pallas as pl
from jax.experimental.pallas import tpu as pltpu
from jax.experimental.pallas import tpu_sc as plsc

NC = 2
NS = 16
NW = NC * NS
L = 16


def _proj_body(x_ref, w_ref, b_ref, o_ref):
    o_ref[...] = (
        jnp.dot(x_ref[...], w_ref[...], preferred_element_type=jnp.float32)
        + b_ref[...]
    )


def _node_proj(x, w, b):
    n, dx = x.shape
    h = w.shape[1]
    blk = 2000
    return pl.pallas_call(
        _proj_body,
        grid=(n // blk,),
        in_specs=[
            pl.BlockSpec((blk, dx), lambda i: (i, 0)),
            pl.BlockSpec((dx, h), lambda i: (0, 0)),
            pl.BlockSpec((1, h), lambda i: (0, 0)),
        ],
        out_specs=pl.BlockSpec((blk, h), lambda i: (i, 0)),
        out_shape=jax.ShapeDtypeStruct((n, h), jnp.float32),
    )(x, w, b.reshape(1, h))


def _gather_body(ew, p_hbm, src_hbm, g_hbm, idx_v, rows_v, sem):
    wid = lax.axis_index("s") * NC + lax.axis_index("c")
    nchunks = ew // 1024

    @pl.loop(0, nchunks)
    def _chunk(ci):
        rowbase = wid * (ew // 128) + ci * 8
        pltpu.sync_copy(src_hbm.at[pl.ds(rowbase, 8)], idx_v)
        copies = [
            pltpu.async_copy(
                p_hbm.at[idx_v.at[j]], rows_v.at[pl.ds(j * 128, 128)], sem
            )
            for j in range(8)
        ]
        for cp in copies:
            cp.wait()
        pltpu.sync_copy(rows_v, g_hbm.at[pl.ds(rowbase * 128, 1024)])


def _sc_gather(p, src2d, e_pad):
    ew = e_pad // NW
    mesh = plsc.VectorSubcoreMesh(
        core_axis_name="c", subcore_axis_name="s", num_cores=NC, num_subcores=NS
    )
    return pl.kernel(
        functools.partial(_gather_body, ew),
        out_type=jax.ShapeDtypeStruct((e_pad, L), jnp.float32),
        mesh=mesh,
        compiler_params=pltpu.CompilerParams(use_tc_tiling_on_sc=False),
        scratch_types=[
            pltpu.VMEM((8, 128), jnp.int32),
            pltpu.VMEM((1024, L), jnp.float32),
            pltpu.SemaphoreType.DMA,
        ],
    )(p, src2d)


def _msg_body(ea_ref, g_ref, w1_ref, w2_ref, b2_ref, o_ref):
    z = jnp.maximum(
        jnp.dot(ea_ref[...], w1_ref[...], preferred_element_type=jnp.float32)
        + g_ref[...],
        0.0,
    )
    o_ref[...] = (
        jnp.dot(z, w2_ref[...], preferred_element_type=jnp.float32) + b2_ref[...]
    )


def _msg_mlp(ea, g, w1e, w2, b2):
    e_pad, de = ea.shape
    msg = w2.shape[1]
    blk = 8192
    return pl.pallas_call(
        _msg_body,
        grid=(e_pad // blk,),
        in_specs=[
            pl.BlockSpec((blk, de), lambda i: (i, 0)),
            pl.BlockSpec((blk, L), lambda i: (i, 0)),
            pl.BlockSpec((de, msg), lambda i: (0, 0)),
            pl.BlockSpec((L, msg), lambda i: (0, 0)),
            pl.BlockSpec((1, msg), lambda i: (0, 0)),
        ],
        out_specs=pl.BlockSpec((blk, msg), lambda i: (i, 0)),
        out_shape=jax.ShapeDtypeStruct((e_pad, msg), jnp.float32),
    )(ea, g, w1e, w2, b2.reshape(1, msg))


_BCAST_DNUMS = lax.GatherDimensionNumbers(
    offset_dims=(), collapsed_slice_dims=(0,), start_index_map=(0,)
)


def _bcast_lane(v, b):
    return lax.gather(
        v,
        jnp.full((L, 1), b, jnp.int32),
        _BCAST_DNUMS,
        slice_sizes=(1,),
        mode=lax.GatherScatterMode.PROMISE_IN_BOUNDS,
    )


def _scatter_body(eg, nh, m_hbm, dst_hbm, out_hbm, acc, d_v, m_v):
    wid = lax.axis_index("s") * NC + lax.axis_index("c")
    g = wid // 2
    h = wid % 2
    col = lax.iota(jnp.int32, L)
    neg_inf = jnp.full((L,), -jnp.inf, dtype=jnp.float32)

    @pl.loop(0, nh + 1)
    def _init(i):
        acc[pl.ds(i * L, L)] = neg_inf

    nchunks = eg // 1024

    @pl.loop(0, nchunks)
    def _chunk(ci):
        ebase = g * eg + ci * 1024
        pltpu.sync_copy(dst_hbm.at[pl.ds(ebase, 1024)], d_v)
        pltpu.sync_copy(m_hbm.at[pl.ds(ebase * L, 1024 * L)], m_v)

        @pl.loop(0, 1024 // L)
        def _vec(i):
            dstv = d_v[pl.ds(i * L, L)]
            off = dstv - h * nh
            owned = (off >= 0) & (off < nh)
            base16 = jnp.where(owned, off, nh) * L
            for b in range(L):
                idx = _bcast_lane(base16, b) + col
                mrow = m_v[pl.ds((i * L + b) * L, L)]
                old = plsc.load_gather(acc, [idx])
                plsc.store_scatter(acc, [idx], jnp.maximum(old, mrow))

    pltpu.sync_copy(acc.at[pl.ds(0, nh * L)], out_hbm.at[h, g])


def _sc_scatter_max(m_flat, dst, e_pad, nh):
    eg = e_pad // (NW // 2)
    mesh = plsc.VectorSubcoreMesh(
        core_axis_name="c", subcore_axis_name="s", num_cores=NC, num_subcores=NS
    )
    return pl.kernel(
        functools.partial(_scatter_body, eg, nh),
        out_type=jax.ShapeDtypeStruct((2, NW // 2, nh * L), jnp.float32),
        mesh=mesh,
        compiler_params=pltpu.CompilerParams(
            use_tc_tiling_on_sc=False, needs_layout_passes=False
        ),
        scratch_types=[
            pltpu.VMEM(((nh + 1) * L,), jnp.float32),
            pltpu.VMEM((1024,), jnp.int32),
            pltpu.VMEM((1024 * L,), jnp.float32),
        ],
    )(m_flat, dst)


def _update_body(x_ref, p_ref, wx_ref, wr_ref, b1_ref, w2_ref, b2_ref, o_ref):
    r = jnp.max(p_ref[0], axis=0)
    r = jnp.where(jnp.isfinite(r), r, 0.0)
    u = jnp.maximum(
        jnp.dot(x_ref[...], wx_ref[...], preferred_element_type=jnp.float32)
        + jnp.dot(r, wr_ref[...], preferred_element_type=jnp.float32)
        + b1_ref[...],
        0.0,
    )
    o_ref[...] = (
        jnp.dot(u, w2_ref[...], preferred_element_type=jnp.float32) + b2_ref[...]
    )


def _update_mlp(x, partials, wx, wr, b1, w2, b2):
    n, dx = x.shape
    hid = wx.shape[1]
    dout = w2.shape[1]
    nh = n // 2
    blk = 1000
    bph = nh // blk
    return pl.pallas_call(
        _update_body,
        grid=(n // blk,),
        in_specs=[
            pl.BlockSpec((blk, dx), lambda i: (i, 0)),
            pl.BlockSpec(
                (1, NW // 2, blk, L), lambda i: (i // bph, 0, i % bph, 0)
            ),
            pl.BlockSpec((dx, hid), lambda i: (0, 0)),
            pl.BlockSpec((L, hid), lambda i: (0, 0)),
            pl.BlockSpec((1, hid), lambda i: (0, 0)),
            pl.BlockSpec((hid, dout), lambda i: (0, 0)),
            pl.BlockSpec((1, dout), lambda i: (0, 0)),
        ],
        out_specs=pl.BlockSpec((blk, dout), lambda i: (i, 0)),
        out_shape=jax.ShapeDtypeStruct((n, dout), jnp.float32),
    )(x, partials, wx, wr, b1.reshape(1, hid), w2, b2.reshape(1, dout))


def kernel(x, edge_index, edge_attr, W_msg1, b_msg1, W_msg2, b_msg2,
           W_udt1, b_udt1, W_udt2, b_udt2):
    n, dx = x.shape
    e = edge_index.shape[1]
    de = edge_attr.shape[1]
    nh = n // 2

    e_pad = -(-e // (NW * 1024)) * (NW * 1024)
    pad = e_pad - e
    src = jnp.concatenate([edge_index[0], jnp.zeros((pad,), jnp.int32)])
    dst = jnp.concatenate([edge_index[1], jnp.full((pad,), n, jnp.int32)])
    ea = jnp.concatenate(
        [edge_attr, jnp.zeros((pad, de), jnp.float32)], axis=0
    )

    p = _node_proj(x, W_msg1[de:], b_msg1)
    g = _sc_gather(p, src.reshape(e_pad // 128, 128), e_pad)
    m = _msg_mlp(ea, g, W_msg1[:de], W_msg2, b_msg2)
    partials = _sc_scatter_max(m.reshape(-1), dst, e_pad, nh)
    partials = partials.reshape(2, NW // 2, nh, L)
    return _update_mlp(x, partials, W_udt1[:dx], W_udt1[dx:], b_udt1,
                       W_udt2, b_udt2)


# speedup vs baseline: 2.8659x; 1.1771x over previous
import functools

import jax
import jax.numpy as jnp
from jax import lax
from jax.experimental import pallas as pl
from jax.experimental.pallas import tpu as pltpu
from jax.experimental.pallas import tpu_sc as plsc

NC = 2
NS = 16
NW = NC * NS
L = 16


def _proj_body(x_ref, w_ref, b_ref, o_ref):
    o_ref[...] = (
        jnp.dot(x_ref[...], w_ref[...], preferred_element_type=jnp.float32)
        + b_ref[...]
    )


def _node_proj(x, w, b):
    n, dx = x.shape
    h = w.shape[1]
    blk = 2000
    return pl.pallas_call(
        _proj_body,
        grid=(n // blk,),
        in_specs=[
            pl.BlockSpec((blk, dx), lambda i: (i, 0)),
            pl.BlockSpec((dx, h), lambda i: (0, 0)),
            pl.BlockSpec((1, h), lambda i: (0, 0)),
        ],
        out_specs=pl.BlockSpec((blk, h), lambda i: (i, 0)),
        out_shape=jax.ShapeDtypeStruct((n, h), jnp.float32),
    )(x, w, b.reshape(1, h))


def _gather_body(ew, p_hbm, src_hbm, g_hbm, idx0, idx1, rows0, rows1,
                 sem_i0, sem_i1, sem_g, sem_w0, sem_w1):
    wid = lax.axis_index("s") * NC + lax.axis_index("c")
    nch = ew // 1000
    idx_b = (idx0, idx1)
    rows_b = (rows0, rows1)
    sem_i = (sem_i0, sem_i1)
    sem_w = (sem_w0, sem_w1)

    def fire_idx(ci, b):
        rowbase = wid * (ew // 125) + ci * 8
        pltpu.async_copy(src_hbm.at[pl.ds(rowbase, 8)], idx_b[b], sem_i[b])

    fire_idx(0, 0)

    @pl.loop(0, nch // 2)
    def _outer(o):
        for b in range(2):
            ci = 2 * o + b

            @pl.when(ci + 1 < nch)
            def _():
                fire_idx(ci + 1, 1 - b)

            pltpu.make_async_copy(
                src_hbm.at[pl.ds(0, 8)], idx_b[b], sem_i[b]
            ).wait()

            @pl.when(o > 0)
            def _():
                pltpu.make_async_copy(
                    rows_b[b], g_hbm.at[pl.ds(0, 1000)], sem_w[b]
                ).wait()

            copies = [
                pltpu.async_copy(
                    p_hbm.at[idx_b[b].at[j]],
                    rows_b[b].at[pl.ds(j * 125, 125)],
                    sem_g,
                )
                for j in range(8)
            ]
            for cp in copies:
                cp.wait()
            pltpu.async_copy(
                rows_b[b], g_hbm.at[pl.ds(wid * ew + ci * 1000, 1000)], sem_w[b]
            )

    for b in range(2):
        pltpu.make_async_copy(
            rows_b[b], g_hbm.at[pl.ds(0, 1000)], sem_w[b]
        ).wait()


def _sc_gather(p, src2d, e):
    ew = e // NW
    mesh = plsc.VectorSubcoreMesh(
        core_axis_name="c", subcore_axis_name="s", num_cores=NC, num_subcores=NS
    )
    return pl.kernel(
        functools.partial(_gather_body, ew),
        out_type=jax.ShapeDtypeStruct((e, L), jnp.float32),
        mesh=mesh,
        compiler_params=pltpu.CompilerParams(use_tc_tiling_on_sc=False),
        scratch_types=[
            pltpu.VMEM((8, 125), jnp.int32),
            pltpu.VMEM((8, 125), jnp.int32),
            pltpu.VMEM((1000, L), jnp.float32),
            pltpu.VMEM((1000, L), jnp.float32),
            pltpu.SemaphoreType.DMA,
            pltpu.SemaphoreType.DMA,
            pltpu.SemaphoreType.DMA,
            pltpu.SemaphoreType.DMA,
            pltpu.SemaphoreType.DMA,
        ],
    )(p, src2d)


def _msg_body(ea_ref, g_ref, w1_ref, w2_ref, b2_ref, o_ref):
    z = jnp.maximum(
        jnp.dot(ea_ref[...], w1_ref[...], preferred_element_type=jnp.float32)
        + g_ref[...],
        0.0,
    )
    o_ref[...] = (
        jnp.dot(z, w2_ref[...], preferred_element_type=jnp.float32) + b2_ref[...]
    )


def _msg_mlp(ea, g, w1e, w2, b2):
    e_pad, de = ea.shape
    msg = w2.shape[1]
    blk = 8000
    return pl.pallas_call(
        _msg_body,
        grid=(e_pad // blk,),
        in_specs=[
            pl.BlockSpec((blk, de), lambda i: (i, 0)),
            pl.BlockSpec((blk, L), lambda i: (i, 0)),
            pl.BlockSpec((de, msg), lambda i: (0, 0)),
            pl.BlockSpec((L, msg), lambda i: (0, 0)),
            pl.BlockSpec((1, msg), lambda i: (0, 0)),
        ],
        out_specs=pl.BlockSpec((blk, msg), lambda i: (i, 0)),
        out_shape=jax.ShapeDtypeStruct((e_pad, msg), jnp.float32),
    )(ea, g, w1e, w2, b2.reshape(1, msg))


_BCAST_DNUMS = lax.GatherDimensionNumbers(
    offset_dims=(), collapsed_slice_dims=(0,), start_index_map=(0,)
)


def _bcast_lane(v, b):
    return lax.gather(
        v,
        jnp.full((L, 1), b, jnp.int32),
        _BCAST_DNUMS,
        slice_sizes=(1,),
        mode=lax.GatherScatterMode.PROMISE_IN_BOUNDS,
    )


def _scatter_body(eg, nh, m_hbm, dst_hbm, out_hbm, acc,
                  d0, d1, m0, m1, sem0, sem1):
    wid = lax.axis_index("s") * NC + lax.axis_index("c")
    g = wid // 2
    h = wid % 2
    col = lax.iota(jnp.int32, L)
    neg_inf = jnp.full((L,), -jnp.inf, dtype=jnp.float32)
    d_b = (d0, d1)
    m_b = (m0, m1)
    sem = (sem0, sem1)

    @pl.loop(0, nh + 1)
    def _init(i):
        acc[pl.ds(i * L, L)] = neg_inf

    nch = eg // 1000
    nvec = 1000 // L + 1

    def fire(ci, b):
        ebase = g * eg + ci * 1000
        pltpu.async_copy(dst_hbm.at[pl.ds(ebase, 1000)], d_b[b], sem[b])
        pltpu.async_copy(m_hbm.at[pl.ds(ebase * L, 1000 * L)], m_b[b], sem[b])

    fire(0, 0)

    @pl.loop(0, nch // 2)
    def _outer(o):
        for b in range(2):
            ci = 2 * o + b

            @pl.when(ci + 1 < nch)
            def _():
                fire(ci + 1, 1 - b)

            pltpu.make_async_copy(
                dst_hbm.at[pl.ds(0, 1000)], d_b[b], sem[b]
            ).wait()
            pltpu.make_async_copy(
                m_hbm.at[pl.ds(0, 1000 * L)], m_b[b], sem[b]
            ).wait()

            @pl.loop(0, nvec)
            def _vec(k):
                eoff = jnp.minimum(k * L, 1000 - L)
                dstv = d_b[b][pl.ds(eoff, L)]
                off = dstv - h * nh
                owned = (off >= 0) & (off < nh)
                base16 = jnp.where(owned, off, nh) * L
                for lane in range(L):
                    idx = _bcast_lane(base16, lane) + col
                    mrow = m_b[b][pl.ds((eoff + lane) * L, L)]
                    old = plsc.load_gather(acc, [idx])
                    plsc.store_scatter(acc, [idx], jnp.maximum(old, mrow))

    pltpu.sync_copy(acc.at[pl.ds(0, nh * L)], out_hbm.at[h, g])


def _sc_scatter_max(m_flat, dst, e, nh):
    eg = e // (NW // 2)
    mesh = plsc.VectorSubcoreMesh(
        core_axis_name="c", subcore_axis_name="s", num_cores=NC, num_subcores=NS
    )
    return pl.kernel(
        functools.partial(_scatter_body, eg, nh),
        out_type=jax.ShapeDtypeStruct((2, NW // 2, nh * L), jnp.float32),
        mesh=mesh,
        compiler_params=pltpu.CompilerParams(
            use_tc_tiling_on_sc=False, needs_layout_passes=False
        ),
        scratch_types=[
            pltpu.VMEM(((nh + 1) * L,), jnp.float32),
            pltpu.VMEM((1000,), jnp.int32),
            pltpu.VMEM((1000,), jnp.int32),
            pltpu.VMEM((1000 * L,), jnp.float32),
            pltpu.VMEM((1000 * L,), jnp.float32),
            pltpu.SemaphoreType.DMA,
            pltpu.SemaphoreType.DMA,
        ],
    )(m_flat, dst)


def _update_body(x_ref, p_ref, wx_ref, wr_ref, b1_ref, w2_ref, b2_ref, o_ref):
    r = jnp.max(p_ref[0], axis=0)
    r = jnp.where(jnp.isfinite(r), r, 0.0)
    u = jnp.maximum(
        jnp.dot(x_ref[...], wx_ref[...], preferred_element_type=jnp.float32)
        + jnp.dot(r, wr_ref[...], preferred_element_type=jnp.float32)
        + b1_ref[...],
        0.0,
    )
    o_ref[...] = (
        jnp.dot(u, w2_ref[...], preferred_element_type=jnp.float32) + b2_ref[...]
    )


def _update_mlp(x, partials, wx, wr, b1, w2, b2):
    n, dx = x.shape
    hid = wx.shape[1]
    dout = w2.shape[1]
    nh = n // 2
    blk = 1000
    bph = nh // blk
    return pl.pallas_call(
        _update_body,
        grid=(n // blk,),
        in_specs=[
            pl.BlockSpec((blk, dx), lambda i: (i, 0)),
            pl.BlockSpec(
                (1, NW // 2, blk, L), lambda i: (i // bph, 0, i % bph, 0)
            ),
            pl.BlockSpec((dx, hid), lambda i: (0, 0)),
            pl.BlockSpec((L, hid), lambda i: (0, 0)),
            pl.BlockSpec((1, hid), lambda i: (0, 0)),
            pl.BlockSpec((hid, dout), lambda i: (0, 0)),
            pl.BlockSpec((1, dout), lambda i: (0, 0)),
        ],
        out_specs=pl.BlockSpec((blk, dout), lambda i: (i, 0)),
        out_shape=jax.ShapeDtypeStruct((n, dout), jnp.float32),
    )(x, partials, wx, wr, b1.reshape(1, hid), w2, b2.reshape(1, dout))


def kernel(x, edge_index, edge_attr, W_msg1, b_msg1, W_msg2, b_msg2,
           W_udt1, b_udt1, W_udt2, b_udt2):
    n, dx = x.shape
    e = edge_index.shape[1]
    de = edge_attr.shape[1]
    nh = n // 2

    src = edge_index[0]
    dst = edge_index[1]

    p = _node_proj(x, W_msg1[de:], b_msg1)
    g = _sc_gather(p, src.reshape(e // 125, 125), e)
    m = _msg_mlp(edge_attr, g, W_msg1[:de], W_msg2, b_msg2)
    partials = _sc_scatter_max(m.reshape(-1), dst, e, nh)
    partials = partials.reshape(2, NW // 2, nh, L)
    return _update_mlp(x, partials, W_udt1[:dx], W_udt1[dx:], b_udt1,
                       W_udt2, b_udt2)


# speedup vs baseline: 5.6852x; 1.9837x over previous
import functools

import jax
import jax.numpy as jnp
from jax import lax
from jax.experimental import pallas as pl
from jax.experimental.pallas import tpu as pltpu
from jax.experimental.pallas import tpu_sc as plsc

NC = 2
NS = 16
NW = NC * NS
L = 16


def _proj_body(x_ref, w_ref, b_ref, o_ref):
    o_ref[...] = (
        jnp.dot(x_ref[...], w_ref[...], preferred_element_type=jnp.float32)
        + b_ref[...]
    )


def _node_proj(x_p, w_bd, b_t):
    np8, dxp = x_p.shape
    hp = w_bd.shape[1]
    return pl.pallas_call(
        _proj_body,
        grid=(1,),
        in_specs=[
            pl.BlockSpec((np8, dxp), lambda i: (0, 0)),
            pl.BlockSpec((dxp, hp), lambda i: (0, 0)),
            pl.BlockSpec((1, hp), lambda i: (0, 0)),
        ],
        out_specs=pl.BlockSpec((np8, hp), lambda i: (0, 0)),
        out_shape=jax.ShapeDtypeStruct((np8, hp), jnp.float32),
    )(x_p, w_bd, b_t)


def _gather_body(ew, p_hbm, src_hbm, g_hbm, idx0, idx1, rows0, rows1,
                 sem_i0, sem_i1, sem_g, sem_w0, sem_w1):
    wid = lax.axis_index("s") * NC + lax.axis_index("c")
    nch = -(-ew // 1024)
    idx_b = (idx0, idx1)
    rows_b = (rows0, rows1)
    sem_i = (sem_i0, sem_i1)
    sem_w = (sem_w0, sem_w1)

    def start_of(ci):
        return wid * ew + jnp.minimum(ci * 1024, ew - 1024)

    def fire_idx(ci, b):
        pltpu.async_copy(
            src_hbm.at[pl.ds(start_of(ci), 1024)], idx_b[b], sem_i[b]
        )

    fire_idx(0, 0)

    @pl.loop(0, nch // 2)
    def _outer(o):
        for b in range(2):
            ci = 2 * o + b

            @pl.when(ci + 1 < nch)
            def _():
                fire_idx(ci + 1, 1 - b)

            pltpu.make_async_copy(
                src_hbm.at[pl.ds(0, 1024)], idx_b[b], sem_i[b]
            ).wait()

            @pl.when(o > 0)
            def _():
                pltpu.make_async_copy(
                    rows_b[b], g_hbm.at[pl.ds(0, 1024)], sem_w[b]
                ).wait()

            copies = [
                pltpu.async_copy(
                    p_hbm.at[idx_b[b].at[pl.ds(j * 128, 128)]],
                    rows_b[b].at[pl.ds(j * 128, 128)],
                    sem_g,
                )
                for j in range(8)
            ]
            for cp in copies:
                cp.wait()
            pltpu.async_copy(
                rows_b[b], g_hbm.at[pl.ds(start_of(ci), 1024)], sem_w[b]
            )

    for b in range(2):
        pltpu.make_async_copy(
            rows_b[b], g_hbm.at[pl.ds(0, 1024)], sem_w[b]
        ).wait()


def _sc_gather(p, src, e):
    ew = e // NW
    mesh = plsc.VectorSubcoreMesh(
        core_axis_name="c", subcore_axis_name="s", num_cores=NC, num_subcores=NS
    )
    return pl.kernel(
        functools.partial(_gather_body, ew),
        out_type=jax.ShapeDtypeStruct((e, L), jnp.float32),
        mesh=mesh,
        compiler_params=pltpu.CompilerParams(use_tc_tiling_on_sc=False),
        scratch_types=[
            pltpu.VMEM((1024,), jnp.int32),
            pltpu.VMEM((1024,), jnp.int32),
            pltpu.VMEM((1024, L), jnp.float32),
            pltpu.VMEM((1024, L), jnp.float32),
            pltpu.SemaphoreType.DMA,
            pltpu.SemaphoreType.DMA,
            pltpu.SemaphoreType.DMA,
            pltpu.SemaphoreType.DMA,
            pltpu.SemaphoreType.DMA,
        ],
    )(p, src)


def _msg_body(ea_ref, g_ref, w1_ref, w2_ref, b2_ref, o_ref):
    z = jnp.maximum(
        jnp.dot(ea_ref[...], w1_ref[...], preferred_element_type=jnp.float32)
        + g_ref[...],
        0.0,
    )
    o_ref[...] = (
        jnp.dot(z, w2_ref[...], preferred_element_type=jnp.float32) + b2_ref[...]
    )


def _msg_mlp(ea_p, g_p, w1_bd, w2_bd, b2_t):
    ep8, dep = ea_p.shape
    blk = 4000
    return pl.pallas_call(
        _msg_body,
        grid=(ep8 // blk,),
        in_specs=[
            pl.BlockSpec((blk, dep), lambda i: (i, 0)),
            pl.BlockSpec((blk, dep), lambda i: (i, 0)),
            pl.BlockSpec((dep, dep), lambda i: (0, 0)),
            pl.BlockSpec((dep, dep), lambda i: (0, 0)),
            pl.BlockSpec((1, dep), lambda i: (0, 0)),
        ],
        out_specs=pl.BlockSpec((blk, dep), lambda i: (i, 0)),
        out_shape=jax.ShapeDtypeStruct((ep8, dep), jnp.float32),
    )(ea_p, g_p, w1_bd, w2_bd, b2_t)


_BCAST_DNUMS = lax.GatherDimensionNumbers(
    offset_dims=(), collapsed_slice_dims=(0,), start_index_map=(0,)
)


def _bcast_lane(v, b):
    return lax.gather(
        v,
        jnp.full((L, 1), b, jnp.int32),
        _BCAST_DNUMS,
        slice_sizes=(1,),
        mode=lax.GatherScatterMode.PROMISE_IN_BOUNDS,
    )


def _scatter_body(eg, nh, m_hbm, dst_hbm, out_hbm, acc,
                  d0, d1, m0, m1, sem0, sem1):
    wid = lax.axis_index("s") * NC + lax.axis_index("c")
    g = wid // 2
    h = wid % 2
    col = lax.iota(jnp.int32, L)
    neg_inf = jnp.full((L,), -jnp.inf, dtype=jnp.float32)
    d_b = (d0, d1)
    m_b = (m0, m1)
    sem = (sem0, sem1)

    @pl.loop(0, nh + 1)
    def _init(i):
        acc[pl.ds(i * L, L)] = neg_inf

    nch = eg // 1000
    nvec = 1000 // L + 1

    def fire(ci, b):
        ebase = g * eg + ci * 1000
        pltpu.async_copy(dst_hbm.at[pl.ds(ebase, 1000)], d_b[b], sem[b])
        pltpu.async_copy(m_hbm.at[pl.ds(ebase * L, 1000 * L)], m_b[b], sem[b])

    fire(0, 0)

    @pl.loop(0, nch // 2)
    def _outer(o):
        for b in range(2):
            ci = 2 * o + b

            @pl.when(ci + 1 < nch)
            def _():
                fire(ci + 1, 1 - b)

            pltpu.make_async_copy(
                dst_hbm.at[pl.ds(0, 1000)], d_b[b], sem[b]
            ).wait()
            pltpu.make_async_copy(
                m_hbm.at[pl.ds(0, 1000 * L)], m_b[b], sem[b]
            ).wait()

            @pl.loop(0, nvec)
            def _vec(k):
                eoff = jnp.minimum(k * L, 1000 - L)
                dstv = d_b[b][pl.ds(eoff, L)]
                off = dstv - h * nh
                owned = (off >= 0) & (off < nh)
                base16 = jnp.where(owned, off, nh) * L
                for lane in range(L):
                    idx = _bcast_lane(base16, lane) + col
                    mrow = m_b[b][pl.ds((eoff + lane) * L, L)]
                    old = plsc.load_gather(acc, [idx])
                    plsc.store_scatter(acc, [idx], jnp.maximum(old, mrow))

    pltpu.sync_copy(acc.at[pl.ds(0, nh * L)], out_hbm.at[h, g])


def _sc_scatter_max(m_flat, dst, e, nh):
    eg = e // (NW // 2)
    mesh = plsc.VectorSubcoreMesh(
        core_axis_name="c", subcore_axis_name="s", num_cores=NC, num_subcores=NS
    )
    return pl.kernel(
        functools.partial(_scatter_body, eg, nh),
        out_type=jax.ShapeDtypeStruct((2, NW // 2, nh * L), jnp.float32),
        mesh=mesh,
        compiler_params=pltpu.CompilerParams(
            use_tc_tiling_on_sc=False, needs_layout_passes=False
        ),
        scratch_types=[
            pltpu.VMEM(((nh + 1) * L,), jnp.float32),
            pltpu.VMEM((1000,), jnp.int32),
            pltpu.VMEM((1000,), jnp.int32),
            pltpu.VMEM((1000 * L,), jnp.float32),
            pltpu.VMEM((1000 * L,), jnp.float32),
            pltpu.SemaphoreType.DMA,
            pltpu.SemaphoreType.DMA,
        ],
    )(m_flat, dst)


def _update_body(x_ref, p_ref, wx_ref, wr_ref, b1_ref, w2_ref, b2_ref, o_ref):
    r = jnp.max(p_ref[0], axis=0)
    r = jnp.where(jnp.isfinite(r), r, 0.0)
    u = jnp.maximum(
        jnp.dot(x_ref[0], wx_ref[...], preferred_element_type=jnp.float32)
        + jnp.dot(r, wr_ref[...], preferred_element_type=jnp.float32)
        + b1_ref[...],
        0.0,
    )
    o_ref[0, ...] = (
        jnp.dot(u, w2_ref[...], preferred_element_type=jnp.float32) + b2_ref[...]
    )


def _update_mlp(x_p3, part_p, wx_bd, wr_bd, b1_t, w2_bd, b2_t):
    _, nhp, dxp = x_p3.shape
    ngrp = part_p.shape[1]
    hp = wr_bd.shape[1]
    dop = w2_bd.shape[1]
    return pl.pallas_call(
        _update_body,
        grid=(2,),
        in_specs=[
            pl.BlockSpec((1, nhp, dxp), lambda i: (i, 0, 0)),
            pl.BlockSpec((1, ngrp, nhp, hp), lambda i: (i, 0, 0, 0)),
            pl.BlockSpec((dxp, hp), lambda i: (0, 0)),
            pl.BlockSpec((hp, hp), lambda i: (0, 0)),
            pl.BlockSpec((1, hp), lambda i: (0, 0)),
            pl.BlockSpec((hp, dop), lambda i: (0, 0)),
            pl.BlockSpec((1, dop), lambda i: (0, 0)),
        ],
        out_specs=pl.BlockSpec((1, nhp, dop), lambda i: (i, 0, 0)),
        out_shape=jax.ShapeDtypeStruct((2, nhp, dop), jnp.float32),
    )(x_p3, part_p, wx_bd, wr_bd, b1_t, w2_bd, b2_t)


def kernel(x, edge_index, edge_attr, W_msg1, b_msg1, W_msg2, b_msg2,
           W_udt1, b_udt1, W_udt2, b_udt2):
    n, dx = x.shape
    e = edge_index.shape[1]
    de = edge_attr.shape[1]
    nh = n // 2

    src = edge_index[0]
    dst = edge_index[1]

    eye8 = jnp.eye(8, dtype=jnp.float32)

    def bd(w):
        return jnp.kron(eye8, w)

    def bt(b):
        return jnp.tile(b, 8)[None, :]

    x_p = x.reshape(n // 8, 8 * dx)
    ea_p = edge_attr.reshape(e // 8, 8 * de)

    p_p = _node_proj(x_p, bd(W_msg1[de:]), bt(b_msg1))
    g = _sc_gather(p_p.reshape(n, L), src, e)
    m_p = _msg_mlp(ea_p, g.reshape(e // 8, 8 * L),
                   bd(W_msg1[:de]), bd(W_msg2), bt(b_msg2))
    partials = _sc_scatter_max(m_p.reshape(-1), dst, e, nh)
    part_p = partials.reshape(2, NW // 2, nh * L // 128, 128)
    x_p3 = x.reshape(2, nh // 8, 8 * dx)
    out_p = _update_mlp(x_p3, part_p, bd(W_udt1[:dx]), bd(W_udt1[dx:]),
                        bt(b_udt1), bd(W_udt2), bt(b_udt2))
    return out_p.reshape(n, W_udt2.shape[1])


# speedup vs baseline: 6.1528x; 1.0823x over previous
import functools

import jax
import jax.numpy as jnp
from jax import lax
from jax.experimental import pallas as pl
from jax.experimental.pallas import tpu as pltpu
from jax.experimental.pallas import tpu_sc as plsc

NC = 2
NS = 16
NW = NC * NS
L = 16


def _proj_body(x_ref, w_ref, b_ref, o_ref):
    o_ref[...] = (
        jnp.dot(x_ref[...], w_ref[...], preferred_element_type=jnp.float32)
        + b_ref[...]
    )


def _node_proj(x_p, w_bd, b_t):
    np8, dxp = x_p.shape
    hp = w_bd.shape[1]
    return pl.pallas_call(
        _proj_body,
        grid=(1,),
        in_specs=[
            pl.BlockSpec((np8, dxp), lambda i: (0, 0)),
            pl.BlockSpec((dxp, hp), lambda i: (0, 0)),
            pl.BlockSpec((1, hp), lambda i: (0, 0)),
        ],
        out_specs=pl.BlockSpec((np8, hp), lambda i: (0, 0)),
        out_shape=jax.ShapeDtypeStruct((np8, hp), jnp.float32),
    )(x_p, w_bd, b_t)


def _banded(ebase):
    k = ebase // 1000
    return (k // 8) * 1000, (k % 8) * L


def _gather_body(ew, p_hbm, src_hbm, g_hbm, idx0, idx1, rows0, rows1,
                 sem_i0, sem_i1, sem_g, sem_w0, sem_w1):
    wid = lax.axis_index("s") * NC + lax.axis_index("c")
    nch = ew // 1000
    idx_b = (idx0, idx1)
    rows_b = (rows0, rows1)
    sem_i = (sem_i0, sem_i1)
    sem_w = (sem_w0, sem_w1)

    def fire_idx(ci, b):
        pltpu.async_copy(
            src_hbm.at[pl.ds(wid * ew + ci * 1000, 1000)], idx_b[b], sem_i[b]
        )

    fire_idx(0, 0)

    @pl.loop(0, nch // 2)
    def _outer(o):
        for b in range(2):
            ci = 2 * o + b

            @pl.when(ci + 1 < nch)
            def _():
                fire_idx(ci + 1, 1 - b)

            pltpu.make_async_copy(
                src_hbm.at[pl.ds(0, 1000)], idx_b[b], sem_i[b]
            ).wait()

            @pl.when(o > 0)
            def _():
                pltpu.make_async_copy(
                    rows_b[b], g_hbm.at[pl.ds(0, 1000), pl.ds(0, L)], sem_w[b]
                ).wait()

            copies = [
                pltpu.async_copy(
                    p_hbm.at[idx_b[b].at[pl.ds(j * 128, min(128, 1000 - j * 128))]],
                    rows_b[b].at[pl.ds(j * 128, min(128, 1000 - j * 128))],
                    sem_g,
                )
                for j in range(8)
            ]
            for cp in copies:
                cp.wait()
            row0, col0 = _banded(wid * ew + ci * 1000)
            pltpu.async_copy(
                rows_b[b],
                g_hbm.at[pl.ds(row0, 1000), pl.ds(col0, L)],
                sem_w[b],
            )

    for b in range(2):
        pltpu.make_async_copy(
            rows_b[b], g_hbm.at[pl.ds(0, 1000), pl.ds(0, L)], sem_w[b]
        ).wait()


def _sc_gather(p, src, e):
    ew = e // NW
    mesh = plsc.VectorSubcoreMesh(
        core_axis_name="c", subcore_axis_name="s", num_cores=NC, num_subcores=NS
    )
    return pl.kernel(
        functools.partial(_gather_body, ew),
        out_type=jax.ShapeDtypeStruct((e // 8, 8 * L), jnp.float32),
        mesh=mesh,
        compiler_params=pltpu.CompilerParams(use_tc_tiling_on_sc=False),
        scratch_types=[
            pltpu.VMEM((1000,), jnp.int32),
            pltpu.VMEM((1000,), jnp.int32),
            pltpu.VMEM((1000, L), jnp.float32),
            pltpu.VMEM((1000, L), jnp.float32),
            pltpu.SemaphoreType.DMA,
            pltpu.SemaphoreType.DMA,
            pltpu.SemaphoreType.DMA,
            pltpu.SemaphoreType.DMA,
            pltpu.SemaphoreType.DMA,
        ],
    )(p, src)


def _msg_body(eat_ref, g_ref, w1_ref, w2_ref, b2_ref, o_ref):
    q = lax.dot_general(
        eat_ref[...], w1_ref[...],
        (((0,), (0,)), ((), ())),
        preferred_element_type=jnp.float32,
    )
    nsub = q.shape[0] // 8000
    subs = [
        jnp.concatenate(
            [q[s * 8000 + j * 1000:s * 8000 + (j + 1) * 1000, :]
             for j in range(8)],
            axis=1,
        )
        for s in range(nsub)
    ]
    qp = jnp.concatenate(subs, axis=0) if nsub > 1 else subs[0]
    z = jnp.maximum(qp + g_ref[...], 0.0)
    o_ref[...] = (
        jnp.dot(z, w2_ref[...], preferred_element_type=jnp.float32) + b2_ref[...]
    )


def _msg_mlp(ea_t, g_pb, w1, w2_bd, b2_t):
    de, e = ea_t.shape
    ep8, dep = g_pb.shape
    blk = 2000
    return pl.pallas_call(
        _msg_body,
        grid=(ep8 // blk,),
        in_specs=[
            pl.BlockSpec((de, 8 * blk), lambda i: (0, i)),
            pl.BlockSpec((blk, dep), lambda i: (i, 0)),
            pl.BlockSpec((de, de), lambda i: (0, 0)),
            pl.BlockSpec((dep, dep), lambda i: (0, 0)),
            pl.BlockSpec((1, dep), lambda i: (0, 0)),
        ],
        out_specs=pl.BlockSpec((blk, dep), lambda i: (i, 0)),
        out_shape=jax.ShapeDtypeStruct((ep8, dep), jnp.float32),
    )(ea_t, g_pb, w1, w2_bd, b2_t)


_BCAST_DNUMS = lax.GatherDimensionNumbers(
    offset_dims=(), collapsed_slice_dims=(0,), start_index_map=(0,)
)


def _bcast_lane(v, b):
    return lax.gather(
        v,
        jnp.full((L, 1), b, jnp.int32),
        _BCAST_DNUMS,
        slice_sizes=(1,),
        mode=lax.GatherScatterMode.PROMISE_IN_BOUNDS,
    )


def _scatter_body(eg, nh, m_hbm, dst_hbm, out_hbm, acc,
                  d0, d1, m0, m1, sem0, sem1):
    wid = lax.axis_index("s") * NC + lax.axis_index("c")
    g = wid // 2
    h = wid % 2
    col = lax.iota(jnp.int32, L)
    neg_inf = jnp.full((L,), -jnp.inf, dtype=jnp.float32)
    d_b = (d0, d1)
    m_b = (m0, m1)
    sem = (sem0, sem1)

    @pl.loop(0, nh + 1)
    def _init(i):
        acc[pl.ds(i * L, L)] = neg_inf

    nch = eg // 1000
    nvec = 1000 // L + 1

    def fire(ci, b):
        ebase = g * eg + ci * 1000
        row0, col0 = _banded(ebase)
        pltpu.async_copy(dst_hbm.at[pl.ds(ebase, 1000)], d_b[b], sem[b])
        pltpu.async_copy(
            m_hbm.at[pl.ds(row0, 1000), pl.ds(col0, L)], m_b[b], sem[b]
        )

    fire(0, 0)

    @pl.loop(0, nch // 2)
    def _outer(o):
        for b in range(2):
            ci = 2 * o + b

            @pl.when(ci + 1 < nch)
            def _():
                fire(ci + 1, 1 - b)

            pltpu.make_async_copy(
                dst_hbm.at[pl.ds(0, 1000)], d_b[b], sem[b]
            ).wait()
            pltpu.make_async_copy(
                m_hbm.at[pl.ds(0, 1000), pl.ds(0, L)], m_b[b], sem[b]
            ).wait()

            @pl.loop(0, nvec)
            def _vec(k):
                eoff = jnp.minimum(k * L, 1000 - L)
                dstv = d_b[b][pl.ds(eoff, L)]
                off = dstv - h * nh
                owned = (off >= 0) & (off < nh)
                base16 = jnp.where(owned, off, nh) * L
                for lane in range(L):
                    idx = _bcast_lane(base16, lane) + col
                    mrow = plsc.load_gather(
                        m_b[b], [jnp.broadcast_to(eoff + lane, (L,)), col]
                    )
                    old = plsc.load_gather(acc, [idx])
                    plsc.store_scatter(acc, [idx], jnp.maximum(old, mrow))

    pltpu.sync_copy(acc.at[pl.ds(0, nh * L)], out_hbm.at[h, g])


def _sc_scatter_max(m_p, dst, e, nh):
    eg = e // (NW // 2)
    mesh = plsc.VectorSubcoreMesh(
        core_axis_name="c", subcore_axis_name="s", num_cores=NC, num_subcores=NS
    )
    return pl.kernel(
        functools.partial(_scatter_body, eg, nh),
        out_type=jax.ShapeDtypeStruct((2, NW // 2, nh * L), jnp.float32),
        mesh=mesh,
        compiler_params=pltpu.CompilerParams(
            use_tc_tiling_on_sc=False, needs_layout_passes=False
        ),
        scratch_types=[
            pltpu.VMEM(((nh + 1) * L,), jnp.float32),
            pltpu.VMEM((1000,), jnp.int32),
            pltpu.VMEM((1000,), jnp.int32),
            pltpu.VMEM((1000, L), jnp.float32),
            pltpu.VMEM((1000, L), jnp.float32),
            pltpu.SemaphoreType.DMA,
            pltpu.SemaphoreType.DMA,
        ],
    )(m_p, dst)


def _update_body(x_ref, p_ref, wx_ref, wr_ref, b1_ref, w2_ref, b2_ref, o_ref):
    r = jnp.max(p_ref[0], axis=0)
    r = jnp.where(jnp.isfinite(r), r, 0.0)
    u = jnp.maximum(
        jnp.dot(x_ref[0], wx_ref[...], preferred_element_type=jnp.float32)
        + jnp.dot(r, wr_ref[...], preferred_element_type=jnp.float32)
        + b1_ref[...],
        0.0,
    )
    o_ref[0, ...] = (
        jnp.dot(u, w2_ref[...], preferred_element_type=jnp.float32) + b2_ref[...]
    )


def _update_mlp(x_p3, part_p, wx_bd, wr_bd, b1_t, w2_bd, b2_t):
    _, nhp, dxp = x_p3.shape
    ngrp = part_p.shape[1]
    hp = wr_bd.shape[1]
    dop = w2_bd.shape[1]
    return pl.pallas_call(
        _update_body,
        grid=(2,),
        in_specs=[
            pl.BlockSpec((1, nhp, dxp), lambda i: (i, 0, 0)),
            pl.BlockSpec((1, ngrp, nhp, hp), lambda i: (i, 0, 0, 0)),
            pl.BlockSpec((dxp, hp), lambda i: (0, 0)),
            pl.BlockSpec((hp, hp), lambda i: (0, 0)),
            pl.BlockSpec((1, hp), lambda i: (0, 0)),
            pl.BlockSpec((hp, dop), lambda i: (0, 0)),
            pl.BlockSpec((1, dop), lambda i: (0, 0)),
        ],
        out_specs=pl.BlockSpec((1, nhp, dop), lambda i: (i, 0, 0)),
        out_shape=jax.ShapeDtypeStruct((2, nhp, dop), jnp.float32),
    )(x_p3, part_p, wx_bd, wr_bd, b1_t, w2_bd, b2_t)


def kernel(x, edge_index, edge_attr, W_msg1, b_msg1, W_msg2, b_msg2,
           W_udt1, b_udt1, W_udt2, b_udt2):
    n, dx = x.shape
    e = edge_index.shape[1]
    de = edge_attr.shape[1]
    nh = n // 2

    src = edge_index[0]
    dst = edge_index[1]

    eye8 = jnp.eye(8, dtype=jnp.float32)

    def bd(w):
        return jnp.kron(eye8, w)

    def bt(b):
        return jnp.tile(b, 8)[None, :]

    x_p = x.reshape(n // 8, 8 * dx)
    ea_t = edge_attr.T

    p_p = _node_proj(x_p, bd(W_msg1[de:]), bt(b_msg1))
    g_pb = _sc_gather(p_p.reshape(n, L), src, e)
    m_p = _msg_mlp(ea_t, g_pb,
                   W_msg1[:de], bd(W_msg2), bt(b_msg2))
    partials = _sc_scatter_max(m_p, dst, e, nh)
    part_p = partials.reshape(2, NW // 2, nh * L // 128, 128)
    x_p3 = x.reshape(2, nh // 8, 8 * dx)
    out_p = _update_mlp(x_p3, part_p, bd(W_udt1[:dx]), bd(W_udt1[dx:]),
                        bt(b_udt1), bd(W_udt2), bt(b_udt2))
    return out_p.reshape(n, W_udt2.shape[1])
